# Initial kernel scaffold; baseline (speedup 1.0000x reference)
import jax
import jax.numpy as jnp
from jax.experimental import pallas as pl


def kernel(feat, edge_index, e_feat, W_self, b_self, W_neigh, b_neigh):
    raise NotImplementedError("write your pallas kernel here")


# speedup vs baseline: 2.2319x; 2.2319x over previous
import jax
import jax.numpy as jnp
from jax import lax
from jax.experimental import pallas as pl
from jax.experimental.pallas import tpu as pltpu
from jax.experimental.pallas import tpu_sc as plsc

N_NODES = 10000
N_EDGES = 320000
D = 128

NC = 2
NS = 16
NW = NC * NS

BATCH = 128
NB = 80
PAD_E = NW * NB * BATCH
DUMP = N_NODES
ACC_R = 10240
DEG_BASE = ACC_R
ACC_T = ACC_R + BATCH
NCHUNK = ACC_T // BATCH


def _sc_body(feat_hbm, src_hbm, dst_hbm, e_hbm, p_out, deg_out,
             src_b, dst_b, e_b, didx_b, rows_v, deg_rows, acc, sem):
    c = lax.axis_index("c")
    s = lax.axis_index("s")
    w = c * NS + s

    zero16 = jnp.zeros((16,), jnp.float32)
    ones16 = jnp.ones((16,), jnp.float32)

    def _zero(i, carry):
        for k in range(D // 16):
            rows_v[i, pl.ds(k * 16, 16)] = zero16
        return carry

    lax.fori_loop(0, BATCH, _zero, 0)

    def _zerod(i, carry):
        for k in range(D // 16):
            deg_rows[i, pl.ds(k * 16, 16)] = zero16
        return carry

    lax.fori_loop(0, BATCH, _zerod, 0)

    for ci in range(NCHUNK):
        @pl.when(s == (ci % NS))
        def _():
            pltpu.sync_copy(rows_v, acc.at[pl.ds(ci * BATCH, BATCH)])

    plsc.subcore_barrier()

    def _batch(j, carry):
        base = (w * NB + j) * BATCH
        pltpu.sync_copy(src_hbm.at[pl.ds(base, BATCH)], src_b)
        pltpu.sync_copy(dst_hbm.at[pl.ds(base, BATCH)], dst_b)
        pltpu.sync_copy(e_hbm.at[pl.ds(base, BATCH)], e_b)
        pltpu.async_copy(feat_hbm.at[src_b], rows_v, sem).wait()

        ii = lax.iota(jnp.int32, 16)

        def _group(g, carry2):
            e16 = e_b[pl.ds(g * 16, 16)]
            dst16 = dst_b[pl.ds(g * 16, 16)]
            didx_b[pl.ds(g * 16, 16)] = DEG_BASE + lax.shift_right_logical(
                dst16, 7)
            dm16 = lax.bitwise_and(dst16, 127)
            for i in range(16):
                r = g * 16 + i
                sel = jnp.full((16,), i, jnp.int32)
                eb = e16.at[sel].get(mode="promise_in_bounds")
                dmb = dm16.at[sel].get(mode="promise_in_bounds")
                for k in range(D // 16):
                    rows_v[r, pl.ds(k * 16, 16)] = (
                        rows_v[r, pl.ds(k * 16, 16)] * eb)
                    deg_rows[r, pl.ds(k * 16, 16)] = jnp.where(
                        ii + (16 * k) == dmb, 1.0, 0.0)
            return carry2

        lax.fori_loop(0, BATCH // 16, _group, 0)

        pltpu.sync_copy(rows_v, acc.at[dst_b], add=True)
        pltpu.sync_copy(deg_rows, acc.at[didx_b], add=True)
        return carry

    lax.fori_loop(0, NB, _batch, 0)

    plsc.subcore_barrier()

    for ci in range(NCHUNK):
        @pl.when(s == (ci % NS))
        def _():
            pltpu.sync_copy(acc.at[pl.ds(ci * BATCH, BATCH)], rows_v)
            if ci < NCHUNK - 1:
                ob = c * ACC_R + ci * BATCH
                pltpu.sync_copy(rows_v, p_out.at[pl.ds(ob, BATCH)])
            else:
                ob = c * BATCH
                pltpu.sync_copy(rows_v, deg_out.at[pl.ds(ob, BATCH)])


_sc_call = pl.kernel(
    _sc_body,
    out_type=(
        jax.ShapeDtypeStruct((NC * ACC_R, D), jnp.float32),
        jax.ShapeDtypeStruct((NC * BATCH, D), jnp.float32),
    ),
    mesh=plsc.VectorSubcoreMesh(core_axis_name="c", subcore_axis_name="s"),
    scratch_types=[
        pltpu.VMEM((BATCH,), jnp.int32),
        pltpu.VMEM((BATCH,), jnp.int32),
        pltpu.VMEM((BATCH,), jnp.float32),
        pltpu.VMEM((BATCH,), jnp.int32),
        pltpu.VMEM((BATCH, D), jnp.float32),
        pltpu.VMEM((BATCH, D), jnp.float32),
        pltpu.VMEM_SHARED((ACC_T, D), jnp.float32),
        pltpu.SemaphoreType.DMA,
    ],
)


ROWS_BLK = 1280
GRID = ACC_R // ROWS_BLK


def _tc_body(feat_ref, p_ref, deg_ref, ws_ref, wn_ref, bs_ref, bn_ref, out_ref):
    p = p_ref[0] + p_ref[1]
    dg = jnp.sum(deg_ref[...], axis=0)[:, None]
    h = p / jnp.maximum(dg, 1.0)
    acc = jnp.dot(feat_ref[...], ws_ref[...],
                  preferred_element_type=jnp.float32,
                  precision=lax.Precision.HIGHEST)
    acc = acc + jnp.dot(h, wn_ref[...],
                        preferred_element_type=jnp.float32,
                        precision=lax.Precision.HIGHEST)
    out_ref[...] = acc + bs_ref[...] + bn_ref[...]


_tc_call = pl.pallas_call(
    _tc_body,
    grid=(GRID,),
    in_specs=[
        pl.BlockSpec((ROWS_BLK, D), lambda i: (i, 0)),
        pl.BlockSpec((NC, ROWS_BLK, D), lambda i: (0, i, 0)),
        pl.BlockSpec((NC, ROWS_BLK), lambda i: (0, i)),
        pl.BlockSpec((D, D), lambda i: (0, 0)),
        pl.BlockSpec((D, D), lambda i: (0, 0)),
        pl.BlockSpec((1, D), lambda i: (0, 0)),
        pl.BlockSpec((1, D), lambda i: (0, 0)),
    ],
    out_specs=pl.BlockSpec((ROWS_BLK, D), lambda i: (i, 0)),
    out_shape=jax.ShapeDtypeStruct((ACC_R, D), jnp.float32),
)


def kernel(feat, edge_index, e_feat, W_self, b_self, W_neigh, b_neigh):
    src = edge_index[0].astype(jnp.int32)
    dst = edge_index[1].astype(jnp.int32)
    ev = e_feat.reshape(-1)

    pad = PAD_E - N_EDGES
    src_p = jnp.pad(src, (0, pad))
    dst_p = jnp.pad(dst, (0, pad), constant_values=DUMP)
    e_p = jnp.pad(ev, (0, pad))

    p_part, deg_part = _sc_call(feat, src_p, dst_p, e_p)
    p_part = p_part.reshape(NC, ACC_R, D)
    deg_part = deg_part.reshape(NC, BATCH * D)[:, :ACC_R]

    feat_pad = jnp.pad(feat, ((0, ACC_R - N_NODES), (0, 0)))
    out = _tc_call(feat_pad, p_part, deg_part, W_self, W_neigh,
                   b_self.reshape(1, D), b_neigh.reshape(1, D))
    return out[:N_NODES]


# speedup vs baseline: 2.3870x; 1.0695x over previous
import jax
import jax.numpy as jnp
from jax import lax
from jax.experimental import pallas as pl
from jax.experimental.pallas import tpu as pltpu
from jax.experimental.pallas import tpu_sc as plsc

N_NODES = 10000
N_EDGES = 320000
D = 128

NC = 2
NS = 16
NW = NC * NS

BATCH = 128
NB = 80
PAD_E = NW * NB * BATCH
DUMP = N_NODES
ACC_R = 10240
DEG_BASE = ACC_R
ACC_T = ACC_R + BATCH
NCHUNK = ACC_T // BATCH


def _sc_body(feat_hbm, src_hbm, dst_hbm, e_hbm, p_out, deg_out,
             src_b, dst_b, e_b, didx_b, rows_v, deg_rows, acc, sem):
    c = lax.axis_index("c")
    s = lax.axis_index("s")
    w = c * NS + s

    zero16 = jnp.zeros((16,), jnp.float32)
    ones16 = jnp.ones((16,), jnp.float32)

    def _zero(i, carry):
        for k in range(D // 16):
            rows_v[i, pl.ds(k * 16, 16)] = zero16
        return carry

    lax.fori_loop(0, BATCH, _zero, 0)

    def _zerod(i, carry):
        for k in range(D // 16):
            deg_rows[i, pl.ds(k * 16, 16)] = zero16
        return carry

    lax.fori_loop(0, BATCH, _zerod, 0)

    for ci in range(NCHUNK):
        @pl.when(s == (ci % NS))
        def _():
            pltpu.sync_copy(rows_v, acc.at[pl.ds(ci * BATCH, BATCH)])

    plsc.subcore_barrier()

    def _batch(j, carry):
        base = (w * NB + j) * BATCH
        a1 = pltpu.async_copy(src_hbm.at[pl.ds(base, BATCH)], src_b, sem)
        a2 = pltpu.async_copy(dst_hbm.at[pl.ds(base, BATCH)], dst_b, sem)
        a3 = pltpu.async_copy(e_hbm.at[pl.ds(base, BATCH)], e_b, sem)
        a1.wait()
        a2.wait()
        a3.wait()
        pltpu.async_copy(feat_hbm.at[src_b], rows_v, sem).wait()

        ii = lax.iota(jnp.int32, 16)

        def _group(g, carry2):
            e16 = e_b[pl.ds(g * 16, 16)]
            dst16 = dst_b[pl.ds(g * 16, 16)]
            didx_b[pl.ds(g * 16, 16)] = DEG_BASE + lax.shift_right_logical(
                dst16, 7)
            dm16 = lax.bitwise_and(dst16, 127)
            for i in range(16):
                r = g * 16 + i
                sel = jnp.full((16,), i, jnp.int32)
                eb = e16.at[sel].get(mode="promise_in_bounds")
                dmb = dm16.at[sel].get(mode="promise_in_bounds")
                for k in range(D // 16):
                    rows_v[r, pl.ds(k * 16, 16)] = (
                        rows_v[r, pl.ds(k * 16, 16)] * eb)
                    deg_rows[r, pl.ds(k * 16, 16)] = jnp.where(
                        ii + (16 * k) == dmb, 1.0, 0.0)
            return carry2

        lax.fori_loop(0, BATCH // 16, _group, 0)

        b1 = pltpu.async_copy(rows_v, acc.at[dst_b], sem, add=True)
        b2 = pltpu.async_copy(deg_rows, acc.at[didx_b], sem, add=True)
        b1.wait()
        b2.wait()
        return carry

    lax.fori_loop(0, NB, _batch, 0)

    plsc.subcore_barrier()

    for ci in range(NCHUNK):
        @pl.when(s == (ci % NS))
        def _():
            pltpu.sync_copy(acc.at[pl.ds(ci * BATCH, BATCH)], rows_v)
            if ci < NCHUNK - 1:
                ob = c * ACC_R + ci * BATCH
                pltpu.sync_copy(rows_v, p_out.at[pl.ds(ob, BATCH)])
            else:
                ob = c * BATCH
                pltpu.sync_copy(rows_v, deg_out.at[pl.ds(ob, BATCH)])


_sc_call = pl.kernel(
    _sc_body,
    out_type=(
        jax.ShapeDtypeStruct((NC * ACC_R, D), jnp.float32),
        jax.ShapeDtypeStruct((NC * BATCH, D), jnp.float32),
    ),
    mesh=plsc.VectorSubcoreMesh(core_axis_name="c", subcore_axis_name="s"),
    scratch_types=[
        pltpu.VMEM((BATCH,), jnp.int32),
        pltpu.VMEM((BATCH,), jnp.int32),
        pltpu.VMEM((BATCH,), jnp.float32),
        pltpu.VMEM((BATCH,), jnp.int32),
        pltpu.VMEM((BATCH, D), jnp.float32),
        pltpu.VMEM((BATCH, D), jnp.float32),
        pltpu.VMEM_SHARED((ACC_T, D), jnp.float32),
        pltpu.SemaphoreType.DMA,
    ],
)


ROWS_BLK = 1280
GRID = ACC_R // ROWS_BLK


def _tc_body(feat_ref, p_ref, deg_ref, ws_ref, wn_ref, bs_ref, bn_ref, out_ref):
    p = p_ref[0] + p_ref[1]
    dg = jnp.sum(deg_ref[...], axis=0)[:, None]
    h = p / jnp.maximum(dg, 1.0)
    acc = jnp.dot(feat_ref[...], ws_ref[...],
                  preferred_element_type=jnp.float32,
                  precision=lax.Precision.HIGHEST)
    acc = acc + jnp.dot(h, wn_ref[...],
                        preferred_element_type=jnp.float32,
                        precision=lax.Precision.HIGHEST)
    out_ref[...] = acc + bs_ref[...] + bn_ref[...]


_tc_call = pl.pallas_call(
    _tc_body,
    grid=(GRID,),
    in_specs=[
        pl.BlockSpec((ROWS_BLK, D), lambda i: (i, 0)),
        pl.BlockSpec((NC, ROWS_BLK, D), lambda i: (0, i, 0)),
        pl.BlockSpec((NC, ROWS_BLK), lambda i: (0, i)),
        pl.BlockSpec((D, D), lambda i: (0, 0)),
        pl.BlockSpec((D, D), lambda i: (0, 0)),
        pl.BlockSpec((1, D), lambda i: (0, 0)),
        pl.BlockSpec((1, D), lambda i: (0, 0)),
    ],
    out_specs=pl.BlockSpec((ROWS_BLK, D), lambda i: (i, 0)),
    out_shape=jax.ShapeDtypeStruct((ACC_R, D), jnp.float32),
)


def kernel(feat, edge_index, e_feat, W_self, b_self, W_neigh, b_neigh):
    src = edge_index[0].astype(jnp.int32)
    dst = edge_index[1].astype(jnp.int32)
    ev = e_feat.reshape(-1)

    pad = PAD_E - N_EDGES
    src_p = jnp.pad(src, (0, pad))
    dst_p = jnp.pad(dst, (0, pad), constant_values=DUMP)
    e_p = jnp.pad(ev, (0, pad))

    p_part, deg_part = _sc_call(feat, src_p, dst_p, e_p)
    p_part = p_part.reshape(NC, ACC_R, D)
    deg_part = deg_part.reshape(NC, BATCH * D)[:, :ACC_R]

    feat_pad = jnp.pad(feat, ((0, ACC_R - N_NODES), (0, 0)))
    out = _tc_call(feat_pad, p_part, deg_part, W_self, W_neigh,
                   b_self.reshape(1, D), b_neigh.reshape(1, D))
    return out[:N_NODES]
